# Initial kernel scaffold; baseline (speedup 1.0000x reference)
import jax
import jax.numpy as jnp
from jax.experimental import pallas as pl


def kernel(inpt, table):
    raise NotImplementedError("write your pallas kernel here")


# speedup vs baseline: 1.7639x; 1.7639x over previous
import functools

import jax
import jax.numpy as jnp
from jax import lax
from jax.experimental import pallas as pl
from jax.experimental.pallas import tpu as pltpu
from jax.experimental.pallas import tpu_sc as plsc

B, S, NSW = 1024, 50, 20
EMB = 64
T = B * S
NC, NS = 2, 16
NW = NC * NS
TPW = T // NW
IDX_COLS = 128
CHUNK_TOK = 32
CHUNK_IDX_ROWS = CHUNK_TOK * NSW // IDX_COLS
NCHUNK = TPW // CHUNK_TOK
ROWS_PER_CHUNK = CHUNK_TOK * NSW


def _sc_kernel(table_hbm, idx_hbm, out_hbm,
               idx_v0, idx_v1, rows_v0, rows_v1, out_v, sem0, sem1):
    wid = lax.axis_index("s") * NC + lax.axis_index("c")
    tok_base = wid * TPW
    idx_base = wid * (TPW * NSW)

    def fire(g, idx_v, rows_v, sem):
        i0 = idx_base + g * ROWS_PER_CHUNK
        pltpu.sync_copy(idx_hbm.at[pl.ds(i0, ROWS_PER_CHUNK)], idx_v)
        for j in range(CHUNK_IDX_ROWS):
            pltpu.async_copy(
                table_hbm.at[idx_v.at[pl.ds(j * IDX_COLS, IDX_COLS)]],
                rows_v.at[pl.ds(j * IDX_COLS, IDX_COLS)],
                sem,
            )

    def drain(rows_v, sem):
        pltpu.make_async_copy(
            table_hbm.at[pl.ds(0, ROWS_PER_CHUNK)], rows_v, sem
        ).wait()

    def compute(g, rows_v):
        @pl.loop(0, CHUNK_TOK)
        def _(t):
            r0 = t * NSW
            for c in range(EMB // 16):
                lanes = pl.ds(c * 16, 16)
                acc = rows_v[r0, lanes]
                for s in range(1, NSW):
                    acc = acc + rows_v[r0 + s, lanes]
                out_v[t, lanes] = acc * (1.0 / NSW)
        pltpu.sync_copy(
            out_v, out_hbm.at[pl.ds(tok_base + g * CHUNK_TOK, CHUNK_TOK)]
        )

    fire(0, idx_v0, rows_v0, sem0)

    @pl.loop(0, NCHUNK, step=2)
    def _(g):
        fire(g + 1, idx_v1, rows_v1, sem1)
        drain(rows_v0, sem0)
        compute(g, rows_v0)

        @pl.when(g + 2 < NCHUNK)
        def _():
            fire(g + 2, idx_v0, rows_v0, sem0)

        drain(rows_v1, sem1)
        compute(g + 1, rows_v1)


@jax.jit
def _pooled_lookup(table, idx2d):
    mesh = plsc.VectorSubcoreMesh(core_axis_name="c", subcore_axis_name="s")
    run = pl.kernel(
        _sc_kernel,
        out_type=jax.ShapeDtypeStruct((T, EMB), jnp.float32),
        mesh=mesh,
        compiler_params=pltpu.CompilerParams(use_tc_tiling_on_sc=False),
        scratch_types=[
            pltpu.VMEM((ROWS_PER_CHUNK,), jnp.int32),
            pltpu.VMEM((ROWS_PER_CHUNK,), jnp.int32),
            pltpu.VMEM((ROWS_PER_CHUNK, EMB), jnp.float32),
            pltpu.VMEM((ROWS_PER_CHUNK, EMB), jnp.float32),
            pltpu.VMEM((CHUNK_TOK, EMB), jnp.float32),
            pltpu.SemaphoreType.DMA,
            pltpu.SemaphoreType.DMA,
        ],
    )
    return run(table, idx2d)


def kernel(inpt, table):
    idx_flat = inpt.reshape(T * NSW)
    out = _pooled_lookup(table, idx_flat)
    return out.reshape(B, S, EMB)
